# baseline (device time: 46671 ns/iter reference)
import jax
import jax.numpy as jnp
from jax import lax
from jax.experimental import pallas as pl
from jax.experimental.pallas import tpu as pltpu

N_DEV = 32
N_LAYERS = 3


def kernel(x, Win0, Wout0, Win1, Wout1, Win2, Wout2):
    b, d = x.shape
    rb = b // N_DEV

    def body(x_ref, win0_ref, wout0_ref, win1_ref, wout1_ref, win2_ref,
             wout2_ref, out_ref, acc_ref, rs_recv_ref, xn_ref,
             rs_send_sems, rs_recv_sems, ag_send_sems, ag_recv_sems):
        my = lax.axis_index("i")

        barrier = pltpu.get_barrier_semaphore()
        for p in range(N_DEV):
            pl.semaphore_signal(
                barrier, inc=1, device_id=(p,),
                device_id_type=pl.DeviceIdType.MESH,
            )

        wins = [win0_ref, win1_ref, win2_ref]
        wouts = [wout0_ref, wout1_ref, wout2_ref]

        xv = x_ref[...].astype(jnp.bfloat16)
        for k in range(N_LAYERS):
            h = jnp.maximum(
                jnp.dot(xv, wins[k][...].astype(jnp.bfloat16),
                        preferred_element_type=jnp.float32),
                0.0,
            )
            acc_ref[...] = jnp.dot(
                h.astype(jnp.bfloat16),
                wouts[k][...].astype(jnp.bfloat16),
                preferred_element_type=jnp.float32,
            ).astype(jnp.bfloat16)

            if k == 0:
                pl.semaphore_wait(barrier, N_DEV)
            rs = []
            for o in range(1, N_DEV):
                tgt = lax.rem(my + o, N_DEV)
                rdma = pltpu.make_async_remote_copy(
                    src_ref=acc_ref.at[pl.ds(tgt * rb, rb)],
                    dst_ref=rs_recv_ref.at[o - 1],
                    send_sem=rs_send_sems.at[o - 1],
                    recv_sem=rs_recv_sems.at[o - 1],
                    device_id=(tgt,),
                    device_id_type=pl.DeviceIdType.MESH,
                )
                rdma.start()
                rs.append(rdma)
            for rdma in rs:
                rdma.wait()
            reduced = acc_ref[pl.ds(my * rb, rb), :].astype(
                jnp.float32
            ) + jnp.sum(rs_recv_ref[...].astype(jnp.float32), axis=0)

            if k < N_LAYERS - 1:
                xn_ref[pl.ds(my * rb, rb), :] = reduced.astype(jnp.bfloat16)
                ag = []
                for o in range(1, N_DEV):
                    tgt = lax.rem(my + o, N_DEV)
                    rdma = pltpu.make_async_remote_copy(
                        src_ref=xn_ref.at[pl.ds(my * rb, rb)],
                        dst_ref=xn_ref.at[pl.ds(my * rb, rb)],
                        send_sem=ag_send_sems.at[o - 1],
                        recv_sem=ag_recv_sems.at[o - 1],
                        device_id=(tgt,),
                        device_id_type=pl.DeviceIdType.MESH,
                    )
                    rdma.start()
                    ag.append(rdma)
                for rdma in ag:
                    rdma.wait()
                xv = xn_ref[...]
            else:
                out_ref[...] = reduced

    return pl.pallas_call(
        body,
        out_shape=jax.ShapeDtypeStruct((rb, d), jnp.float32),
        in_specs=[pl.BlockSpec(memory_space=pltpu.VMEM)] * 7,
        out_specs=pl.BlockSpec(memory_space=pltpu.VMEM),
        scratch_shapes=[
            pltpu.VMEM((b, d), jnp.bfloat16),
            pltpu.VMEM((N_DEV - 1, rb, d), jnp.bfloat16),
            pltpu.VMEM((b, d), jnp.bfloat16),
            pltpu.SemaphoreType.DMA((N_DEV - 1,)),
            pltpu.SemaphoreType.DMA((N_DEV - 1,)),
            pltpu.SemaphoreType.DMA((N_DEV - 1,)),
            pltpu.SemaphoreType.DMA((N_DEV - 1,)),
        ],
        compiler_params=pltpu.CompilerParams(collective_id=0),
    )(x, Win0, Wout0, Win1, Wout1, Win2, Wout2)
